# baseline (speedup 1.0000x reference)
import functools

import jax
import jax.numpy as jnp
from jax import lax
from jax.experimental import pallas as pl
from jax.experimental.pallas import tpu as pltpu
from jax.experimental.pallas import tpu_sc as plsc

_NC = 2
_NS = 16
_NW = _NC * _NS


def _user_proj(eut, w128):
    D, V = eut.shape
    G = 1000
    e3 = eut.reshape(D, G, G)

    def body(e_r, w_r, o_r):
        o_r[...] = lax.dot_general(e_r[...], w_r[...],
                                   (((0,), (0,)), ((), ())),
                                   preferred_element_type=jnp.float32)

    p3 = pl.pallas_call(
        body,
        grid=(G // 8,),
        in_specs=[pl.BlockSpec((D, 8, G), lambda i: (0, i, 0)),
                  pl.BlockSpec(w128.shape, lambda i: (0, 0))],
        out_specs=pl.BlockSpec((8, G, 128), lambda i: (i, 0, 0)),
        out_shape=jax.ShapeDtypeStruct((G, G, 128), jnp.float32),
    )(e3, w128)
    return p3.reshape(V, 128)


def _sc_gather(em_rm, hist2, pu, eg, user_ids, movie_ids, genre_ids):
    D = em_rm.shape[1]
    R = hist2.shape[1]
    B = user_ids.shape[0]
    bpw = B // _NW
    ng = hist2.shape[0] // _NW
    NB = 10
    LOOK = 8
    mesh = plsc.VectorSubcoreMesh(core_axis_name="c", subcore_axis_name="s",
                                  num_cores=_NC, num_subcores=_NS)

    @functools.partial(
        pl.kernel,
        out_type=(
            jax.ShapeDtypeStruct((B, 128), jnp.float32),
            jax.ShapeDtypeStruct((hist2.shape[0], R, D), jnp.float32),
            jax.ShapeDtypeStruct((B, D), jnp.float32),
            jax.ShapeDtypeStruct((B, D), jnp.float32),
        ),
        mesh=mesh,
        scratch_types=[
            pltpu.VMEM((bpw,), jnp.int32),
            pltpu.VMEM((bpw, D), jnp.float32),
            pltpu.VMEM((bpw, 128), jnp.float32),
            pltpu.VMEM((ng, R), jnp.int32),
            pltpu.VMEM((NB, R, D), jnp.float32),
            pltpu.SemaphoreType.DMA,
            pltpu.SemaphoreType.DMA,
        ],
        compiler_params=pltpu.CompilerParams(use_tc_tiling_on_sc=False),
    )
    def k(em, hid, put, egt, uid, mid, gid, us_o, uh_o, im_o, ig_o,
          idx_s, val_s, valp_s, hidx, hbuf, gsem, wsem):
        w = lax.axis_index("s") * _NC + lax.axis_index("c")
        g0 = w * ng
        base = w * bpw

        pltpu.sync_copy(hid.at[pl.ds(g0, ng)], hidx)

        def g_start(g, slot):
            return pltpu.async_copy(em.at[hidx.at[g]], hbuf.at[slot], gsem)

        def g_wait(g, slot):
            pltpu.make_async_copy(em.at[hidx.at[g]], hbuf.at[slot], gsem).wait()

        def w_start(g, slot):
            return pltpu.async_copy(hbuf.at[slot], uh_o.at[g0 + g], wsem)

        def w_wait(g, slot):
            pltpu.make_async_copy(hbuf.at[slot], uh_o.at[g0 + g], wsem).wait()

        for b in range(LOOK):
            g_start(b, b)

        @pl.loop(0, ng // NB)
        def _(i):
            for b in range(NB):
                g = i * NB + b

                @pl.when(g >= 2)
                def _():
                    w_wait(g - 2, (b - 2) % NB)

                @pl.when(g + LOOK < ng)
                def _():
                    g_start(g + LOOK, (b + LOOK) % NB)

                g_wait(g, b)
                w_start(g, b)

        w_wait(ng - 2, (ng - 2) % NB)
        w_wait(ng - 1, (ng - 1) % NB)

        pltpu.sync_copy(uid.at[pl.ds(base, bpw)], idx_s)
        pltpu.async_copy(put.at[idx_s], valp_s, gsem).wait()
        pltpu.sync_copy(valp_s, us_o.at[pl.ds(base, bpw)])
        for ids_hbm, table, out in ((mid, em, im_o), (gid, egt, ig_o)):
            pltpu.sync_copy(ids_hbm.at[pl.ds(base, bpw)], idx_s)
            pltpu.async_copy(table.at[idx_s], val_s, gsem).wait()
            pltpu.sync_copy(val_s, out.at[pl.ds(base, bpw)])

    return k(em_rm, hist2, pu, eg, user_ids, movie_ids, genre_ids)


def _tc_towers(us, uh, im, ig, Wu1b, bu1, Wu2, bu2,
               Wi1a, Wi1b, bi1, Wi2, bi2):
    B = uh.shape[0]
    BLK = 512

    def body(us_r, uh_r, im_r, ig_r, wu1b_r, bu1_r, wu2_r, bu2_r,
             wi1a_r, wi1b_r, bi1_r, wi2_r, bi2_r, o_r):
        f32 = jnp.float32
        hu = us_r[...][:, :64]
        hu += jnp.dot(uh_r[...], wu1b_r[...], preferred_element_type=f32)
        hu = jnp.maximum(hu + bu1_r[...], 0.0)
        uo = jnp.dot(hu, wu2_r[...], preferred_element_type=f32) + bu2_r[...]
        hi = jnp.dot(im_r[...], wi1a_r[...], preferred_element_type=f32)
        hi += jnp.dot(ig_r[...], wi1b_r[...], preferred_element_type=f32)
        hi = jnp.maximum(hi + bi1_r[...], 0.0)
        io = jnp.dot(hi, wi2_r[...], preferred_element_type=f32) + bi2_r[...]
        o_r[...] = jax.nn.sigmoid(jnp.sum(uo * io, axis=1))

    def row_spec(arr):
        return pl.BlockSpec((BLK, arr.shape[1]), lambda i: (i, 0))

    def full_spec(arr):
        return pl.BlockSpec(arr.shape, lambda i: (0,) * arr.ndim)

    args = (us, uh, im, ig, Wu1b, bu1, Wu2, bu2,
            Wi1a, Wi1b, bi1, Wi2, bi2)
    specs = [row_spec(us), row_spec(uh), row_spec(im), row_spec(ig)] + [
        full_spec(a) for a in args[4:]
    ]
    return pl.pallas_call(
        body,
        grid=(B // BLK,),
        in_specs=specs,
        out_specs=pl.BlockSpec((BLK,), lambda i: (i,)),
        out_shape=jax.ShapeDtypeStruct((B,), jnp.float32),
    )(*args)


def kernel(E_user, E_movie, E_genre, Wu1, bu1, Wu2, bu2, Wi1, bi1, Wi2, bi2,
           user_ids, hist_ids, movie_ids, genre_ids):
    B, L = hist_ids.shape
    V, D = E_movie.shape
    hist2 = hist_ids.astype(jnp.int32).reshape(B * L // 128, 128)
    w128 = jnp.zeros((D, 128), jnp.float32).at[:, :64].set(Wu1[:D])
    pu = _user_proj(E_user.T, w128)
    us, uh, im, ig = _sc_gather(
        E_movie, hist2, pu, E_genre,
        user_ids.astype(jnp.int32), movie_ids.astype(jnp.int32),
        genre_ids.astype(jnp.int32))
    uh2 = uh.reshape(B, L * D)
    return _tc_towers(us, uh2, im, ig,
                      Wu1[D:], bu1[None], Wu2, bu2[None],
                      Wi1[:D], Wi1[D:], bi1[None], Wi2, bi2[None])


# speedup vs baseline: 1.0358x; 1.0358x over previous
import functools

import jax
import jax.numpy as jnp
from jax import lax
from jax.experimental import pallas as pl
from jax.experimental.pallas import tpu as pltpu
from jax.experimental.pallas import tpu_sc as plsc

_NC = 2
_NS = 16
_NW = _NC * _NS


def _sc_gather(em_rm, hist2, eu, eg, user_ids, movie_ids, genre_ids):
    D = em_rm.shape[1]
    R = hist2.shape[1]
    B = user_ids.shape[0]
    bpw = B // _NW
    ng = hist2.shape[0] // _NW
    NB = 10
    LOOK = 8
    mesh = plsc.VectorSubcoreMesh(core_axis_name="c", subcore_axis_name="s",
                                  num_cores=_NC, num_subcores=_NS)

    @functools.partial(
        pl.kernel,
        out_type=(
            jax.ShapeDtypeStruct((B, D), jnp.float32),
            jax.ShapeDtypeStruct((hist2.shape[0], R, D), jnp.float32),
            jax.ShapeDtypeStruct((B, D), jnp.float32),
            jax.ShapeDtypeStruct((B, D), jnp.float32),
        ),
        mesh=mesh,
        scratch_types=[
            pltpu.VMEM((bpw,), jnp.int32),
            pltpu.VMEM((bpw, D), jnp.float32),
            pltpu.VMEM((ng, R), jnp.int32),
            pltpu.VMEM((NB, R, D), jnp.float32),
            pltpu.SemaphoreType.DMA,
            pltpu.SemaphoreType.DMA,
        ],
        compiler_params=pltpu.CompilerParams(use_tc_tiling_on_sc=False),
    )
    def k(em, hid, eut, egt, uid, mid, gid, us_o, uh_o, im_o, ig_o,
          idx_s, val_s, hidx, hbuf, gsem, wsem):
        w = lax.axis_index("s") * _NC + lax.axis_index("c")
        g0 = w * ng
        base = w * bpw

        pltpu.sync_copy(hid.at[pl.ds(g0, ng)], hidx)

        def g_start(g, slot):
            return pltpu.async_copy(em.at[hidx.at[g]], hbuf.at[slot], gsem)

        def g_wait(g, slot):
            pltpu.make_async_copy(em.at[hidx.at[g]], hbuf.at[slot], gsem).wait()

        def w_start(g, slot):
            return pltpu.async_copy(hbuf.at[slot], uh_o.at[g0 + g], wsem)

        def w_wait(g, slot):
            pltpu.make_async_copy(hbuf.at[slot], uh_o.at[g0 + g], wsem).wait()

        for b in range(LOOK):
            g_start(b, b)

        @pl.loop(0, ng // NB)
        def _(i):
            for b in range(NB):
                g = i * NB + b

                @pl.when(g >= 2)
                def _():
                    w_wait(g - 2, (b - 2) % NB)

                @pl.when(g + LOOK < ng)
                def _():
                    g_start(g + LOOK, (b + LOOK) % NB)

                g_wait(g, b)
                w_start(g, b)

        w_wait(ng - 2, (ng - 2) % NB)
        w_wait(ng - 1, (ng - 1) % NB)

        for ids_hbm, table, out in ((uid, eut, us_o), (mid, em, im_o),
                                    (gid, egt, ig_o)):
            pltpu.sync_copy(ids_hbm.at[pl.ds(base, bpw)], idx_s)
            pltpu.async_copy(table.at[idx_s], val_s, gsem).wait()
            pltpu.sync_copy(val_s, out.at[pl.ds(base, bpw)])

    return k(em_rm, hist2, eu, eg, user_ids, movie_ids, genre_ids)


def _tc_towers(us, uh, im, ig, Wu1a, Wu1b, bu1, Wu2, bu2,
               Wi1a, Wi1b, bi1, Wi2, bi2):
    B = uh.shape[0]
    BLK = 512

    def body(us_r, uh_r, im_r, ig_r, wu1a_r, wu1b_r, bu1_r, wu2_r, bu2_r,
             wi1a_r, wi1b_r, bi1_r, wi2_r, bi2_r, o_r):
        f32 = jnp.float32
        hu = jnp.dot(us_r[...], wu1a_r[...], preferred_element_type=f32)
        hu += jnp.dot(uh_r[...], wu1b_r[...], preferred_element_type=f32)
        hu = jnp.maximum(hu + bu1_r[...], 0.0)
        uo = jnp.dot(hu, wu2_r[...], preferred_element_type=f32) + bu2_r[...]
        hi = jnp.dot(im_r[...], wi1a_r[...], preferred_element_type=f32)
        hi += jnp.dot(ig_r[...], wi1b_r[...], preferred_element_type=f32)
        hi = jnp.maximum(hi + bi1_r[...], 0.0)
        io = jnp.dot(hi, wi2_r[...], preferred_element_type=f32) + bi2_r[...]
        o_r[...] = jax.nn.sigmoid(jnp.sum(uo * io, axis=1))

    def row_spec(arr):
        return pl.BlockSpec((BLK, arr.shape[1]), lambda i: (i, 0))

    def full_spec(arr):
        return pl.BlockSpec(arr.shape, lambda i: (0,) * arr.ndim)

    args = (us, uh, im, ig, Wu1a, Wu1b, bu1, Wu2, bu2,
            Wi1a, Wi1b, bi1, Wi2, bi2)
    specs = [row_spec(us), row_spec(uh), row_spec(im), row_spec(ig)] + [
        full_spec(a) for a in args[4:]
    ]
    return pl.pallas_call(
        body,
        grid=(B // BLK,),
        in_specs=specs,
        out_specs=pl.BlockSpec((BLK,), lambda i: (i,)),
        out_shape=jax.ShapeDtypeStruct((B,), jnp.float32),
    )(*args)


def kernel(E_user, E_movie, E_genre, Wu1, bu1, Wu2, bu2, Wi1, bi1, Wi2, bi2,
           user_ids, hist_ids, movie_ids, genre_ids):
    B, L = hist_ids.shape
    V, D = E_movie.shape
    hist2 = hist_ids.astype(jnp.int32).reshape(B * L // 128, 128)
    us, uh, im, ig = _sc_gather(
        E_movie, hist2, E_user, E_genre,
        user_ids.astype(jnp.int32), movie_ids.astype(jnp.int32),
        genre_ids.astype(jnp.int32))
    uh2 = uh.reshape(B, L * D)
    return _tc_towers(us, uh2, im, ig,
                      Wu1[:D], Wu1[D:], bu1[None], Wu2, bu2[None],
                      Wi1[:D], Wi1[D:], bi1[None], Wi2, bi2[None])
